# Initial kernel scaffold; baseline (speedup 1.0000x reference)
import jax
import jax.numpy as jnp
from jax.experimental import pallas as pl


def kernel(pos, node_size_x, node_size_y, bin_center_x, bin_center_y):
    raise NotImplementedError("write your pallas kernel here")


# speedup vs baseline: 66.0374x; 66.0374x over previous
import math

import jax
import jax.numpy as jnp
from jax.experimental import pallas as pl
from jax.experimental.pallas import tpu as pltpu

_NMOV = 500000
_NTERM = 10000
_NFIL = 190000
_NTOT = _NMOV + _NTERM + _NFIL
_NBX = 512
_NBY = 512
_XL, _YL, _XH, _YH = 0.0, 0.0, 1000.0, 1000.0
_BSX = (_XH - _XL) / _NBX
_BSY = (_YH - _YL) / _NBY
_TD = 0.9
_SQRT2 = math.sqrt(2.0)
_BIN_AREA = _BSX * _BSY

_C = 2048
_STEPS = (_NTOT + _C - 1) // _C
_NPAD = _C * _STEPS


def _dmap_kernel(x_ref, y_ref, sx_ref, sy_ref, w_ref,
                 cost_ref, maxd_ref, acc_ref):
    k = pl.program_id(0)
    x = x_ref[0, 0, :]
    y = y_ref[0, 0, :]
    sx = sx_ref[0, 0, :]
    sy = sy_ref[0, 0, :]
    w = w_ref[0, 0, :]

    blo_x = jax.lax.broadcasted_iota(
        jnp.int32, (_NBX, 1), 0).astype(jnp.float32) * _BSX
    blo_y = jax.lax.broadcasted_iota(
        jnp.int32, (_NBY, 1), 0).astype(jnp.float32) * _BSY
    px = jnp.minimum(x + sx, blo_x + _BSX) - jnp.maximum(x, blo_x)
    py = jnp.minimum(y + sy, blo_y + _BSY) - jnp.maximum(y, blo_y)
    u = (jnp.maximum(px, 0.0) * w).astype(jnp.bfloat16)
    v = jnp.maximum(py, 0.0).astype(jnp.bfloat16)
    part = jax.lax.dot_general(u, v, (((1,), (1,)), ((), ())),
                               preferred_element_type=jnp.float32)

    @pl.when(k == 0)
    def _init():
        acc_ref[:, :] = part

    @pl.when(k > 0)
    def _acc():
        acc_ref[:, :] = acc_ref[:, :] + part

    @pl.when(k == _STEPS - 1)
    def _fin():
        d = acc_ref[:, :]
        cost_ref[:, :] = jnp.sum(jnp.maximum(d - _TD * _BIN_AREA, 0.0),
                                 keepdims=True)
        maxd_ref[:, :] = jnp.max(d, keepdims=True) / _BIN_AREA


def kernel(pos, node_size_x, node_size_y, bin_center_x, bin_center_y):
    pos_x = pos[:_NTOT]
    pos_y = pos[_NTOT:]
    sxc = jnp.maximum(node_size_x, _BSX * _SQRT2)
    syc = jnp.maximum(node_size_y, _BSY * _SQRT2)
    offx = (node_size_x - sxc) * 0.5
    offy = (node_size_y - syc) * 0.5
    area = node_size_x * node_size_y
    ratio = area / (sxc * syc)
    mean_area = jnp.mean(area[:_NMOV]) * 10.0
    row_h = jnp.min(node_size_y[:_NMOV]) * 2.0
    macro = (area[:_NMOV] > mean_area) & (node_size_y[:_NMOV] > row_h)
    ratio_mov = jnp.where(macro, _TD, ratio[:_NMOV])

    mov = slice(0, _NMOV)
    fix = slice(_NMOV, _NMOV + _NTERM)
    fil = slice(_NMOV + _NTERM, _NTOT)
    x_all = jnp.concatenate([pos_x[mov] + offx[mov], pos_x[fix],
                             pos_x[fil] + offx[fil]])
    y_all = jnp.concatenate([pos_y[mov] + offy[mov], pos_y[fix],
                             pos_y[fil] + offy[fil]])
    sx_all = jnp.concatenate([sxc[mov], node_size_x[fix], sxc[fil]])
    sy_all = jnp.concatenate([syc[mov], node_size_y[fix], syc[fil]])
    w_all = jnp.concatenate([ratio_mov, jnp.full((_NTERM,), _TD, jnp.float32),
                             ratio[fil]])

    pad = _NPAD - _NTOT
    def _prep(a):
        return jnp.pad(a, (0, pad)).reshape(_STEPS, 1, _C)

    ins = [_prep(a) for a in (x_all, y_all, sx_all, sy_all, w_all)]

    spec = pl.BlockSpec((1, 1, _C), lambda k: (k, 0, 0))
    out_spec = pl.BlockSpec((1, 1), lambda k: (0, 0))
    cost, maxd = pl.pallas_call(
        _dmap_kernel,
        grid=(_STEPS,),
        in_specs=[spec] * 5,
        out_specs=[out_spec, out_spec],
        out_shape=[jax.ShapeDtypeStruct((1, 1), jnp.float32),
                   jax.ShapeDtypeStruct((1, 1), jnp.float32)],
        scratch_shapes=[pltpu.VMEM((_NBX, _NBY), jnp.float32)],
    )(*ins)
    return (cost.reshape(()), maxd.reshape(()))


# speedup vs baseline: 108.4723x; 1.6426x over previous
import functools
import math

import jax
import jax.numpy as jnp
from jax import lax
from jax.experimental import pallas as pl
from jax.experimental.pallas import tpu as pltpu
from jax.experimental.pallas import tpu_sc as plsc

_NMOV = 500000
_NTERM = 10000
_NFIL = 190000
_NTOT = _NMOV + _NTERM + _NFIL
_NMF = _NMOV + _NFIL
_NB = 512
_NBINS = _NB * _NB
_XL, _YL, _XH, _YH = 0.0, 0.0, 1000.0, 1000.0
_BS = (_XH - _XL) / _NB
_INV_BS = 1.0 / _BS
_TD = 0.9
_SQRT2 = math.sqrt(2.0)
_BIN_AREA = _BS * _BS

_NC = 2
_NS = 16
_NW = _NC * _NS
_L = 16

_CH = 1024
_CPW = 22
_MV_PER_W = _CPW * _CH
_NMF_PAD = _NW * _MV_PER_W

_FX_PER_W = 384
_NF_PAD = _NW * _FX_PER_W
_FXV = _FX_PER_W // _L

_SLICE = _NBINS // _NS


def _win_overlap(p, size, k_off):
    bi = jnp.clip((p * _INV_BS).astype(jnp.int32), 0, _NB - 1)
    bw = bi + k_off
    valid = bw < _NB
    bc = jnp.minimum(bw, _NB - 1)
    blo = bc.astype(jnp.float32) * _BS
    ov = jnp.minimum(p + size, blo + _BS) - jnp.maximum(p, blo)
    ov = jnp.where(valid, jnp.maximum(ov, 0.0), 0.0)
    return ov, bc


def _sc_body(xm, ym, sxm, sym, wm, xf, yf, sxf, syf, wf, zeros_hbm,
             out_maps,
             xb, yb, sxb, syb, wb, vals, idx2,
             fxb, fyb, fsxb, fsyb, fwb, fvals, fidx2,
             dmap):
    cid = lax.axis_index("c")
    sid = lax.axis_index("s")
    wid = cid * _NS + sid

    pltpu.sync_copy(zeros_hbm.at[pl.ds(sid * _SLICE, _SLICE)],
                    dmap.at[pl.ds(sid * _SLICE, _SLICE)])
    plsc.subcore_barrier()

    def chunk_body(ch, _):
        base = wid * _MV_PER_W + ch * _CH
        pltpu.sync_copy(xm.at[pl.ds(base, _CH)], xb)
        pltpu.sync_copy(ym.at[pl.ds(base, _CH)], yb)
        pltpu.sync_copy(sxm.at[pl.ds(base, _CH)], sxb)
        pltpu.sync_copy(sym.at[pl.ds(base, _CH)], syb)
        pltpu.sync_copy(wm.at[pl.ds(base, _CH)], wb)

        def vec_body(v, _):
            o = v * _L
            x = xb[pl.ds(o, _L)]
            y = yb[pl.ds(o, _L)]
            sx = sxb[pl.ds(o, _L)]
            sy = syb[pl.ds(o, _L)]
            w = wb[pl.ds(o, _L)]
            pxs = []
            pys = []
            for k in range(3):
                px, bxc = _win_overlap(x, sx, k)
                pxs.append((px * w, bxc * _NB))
                py, byc = _win_overlap(y, sy, k)
                pys.append((py, byc))
            for i in range(3):
                pxw, bx5 = pxs[i]
                for j in range(3):
                    py, byc = pys[j]
                    k9 = i * 3 + j
                    vals[pl.ds(k9 * _CH + o, _L)] = pxw * py
                    idx2[pl.ds(k9 * _CH + o, _L)] = bx5 + byc
            return _

        lax.fori_loop(0, _CH // _L, vec_body, None)
        pltpu.sync_copy(vals, dmap.at[idx2], add=True)
        return _

    lax.fori_loop(0, _CPW, chunk_body, None)

    fbase = wid * _FX_PER_W
    pltpu.sync_copy(xf.at[pl.ds(fbase, _FX_PER_W)], fxb)
    pltpu.sync_copy(yf.at[pl.ds(fbase, _FX_PER_W)], fyb)
    pltpu.sync_copy(sxf.at[pl.ds(fbase, _FX_PER_W)], fsxb)
    pltpu.sync_copy(syf.at[pl.ds(fbase, _FX_PER_W)], fsyb)
    pltpu.sync_copy(wf.at[pl.ds(fbase, _FX_PER_W)], fwb)

    def fvec_body(v, _):
        o = v * _L
        x = fxb[pl.ds(o, _L)]
        y = fyb[pl.ds(o, _L)]
        sx = fsxb[pl.ds(o, _L)]
        sy = fsyb[pl.ds(o, _L)]
        w = fwb[pl.ds(o, _L)]
        pxs = []
        pys = []
        for k in range(8):
            px, bxc = _win_overlap(x, sx, k)
            pxs.append((px * w, bxc * _NB))
            py, byc = _win_overlap(y, sy, k)
            pys.append((py, byc))
        for i in range(8):
            pxw, bx5 = pxs[i]
            for j in range(8):
                py, byc = pys[j]
                k64 = i * 8 + j
                p = k64 * _FX_PER_W + o
                fvals[pl.ds(p, _L)] = pxw * py
                fidx2[pl.ds(p, _L)] = bx5 + byc
        return _

    lax.fori_loop(0, _FXV, fvec_body, None)
    pltpu.sync_copy(fvals, dmap.at[fidx2], add=True)

    plsc.subcore_barrier()
    pltpu.sync_copy(dmap.at[pl.ds(sid * _SLICE, _SLICE)],
                    out_maps.at[cid, pl.ds(sid * _SLICE, _SLICE)])


def _sc_density_maps(xm, ym, sxm, sym, wm, xf, yf, sxf, syf, wf, zeros):
    mesh = plsc.VectorSubcoreMesh(core_axis_name="c", subcore_axis_name="s",
                                  num_cores=_NC, num_subcores=_NS)
    f = pl.kernel(
        _sc_body,
        out_type=jax.ShapeDtypeStruct((_NC, _NBINS), jnp.float32),
        mesh=mesh,
        scratch_types=[
            pltpu.VMEM((_CH,), jnp.float32),
            pltpu.VMEM((_CH,), jnp.float32),
            pltpu.VMEM((_CH,), jnp.float32),
            pltpu.VMEM((_CH,), jnp.float32),
            pltpu.VMEM((_CH,), jnp.float32),
            pltpu.VMEM((9 * _CH,), jnp.float32),
            pltpu.VMEM((9 * _CH,), jnp.int32),
            pltpu.VMEM((_FX_PER_W,), jnp.float32),
            pltpu.VMEM((_FX_PER_W,), jnp.float32),
            pltpu.VMEM((_FX_PER_W,), jnp.float32),
            pltpu.VMEM((_FX_PER_W,), jnp.float32),
            pltpu.VMEM((_FX_PER_W,), jnp.float32),
            pltpu.VMEM((64 * _FX_PER_W,), jnp.float32),
            pltpu.VMEM((64 * _FX_PER_W,), jnp.int32),
            pltpu.VMEM_SHARED((_NBINS,), jnp.float32),
        ],
    )
    return f(xm, ym, sxm, sym, wm, xf, yf, sxf, syf, wf, zeros)


def _reduce_kernel(maps_ref, cost_ref, maxd_ref):
    a = maps_ref[:, :]
    d = a[:_NB, :] + a[_NB:, :]
    cost_ref[:, :] = jnp.sum(jnp.maximum(d - _TD * _BIN_AREA, 0.0),
                             keepdims=True)
    maxd_ref[:, :] = jnp.max(d, keepdims=True) / _BIN_AREA


def kernel(pos, node_size_x, node_size_y, bin_center_x, bin_center_y):
    pos_x = pos[:_NTOT]
    pos_y = pos[_NTOT:]
    sxc = jnp.maximum(node_size_x, _BS * _SQRT2)
    syc = jnp.maximum(node_size_y, _BS * _SQRT2)
    offx = (node_size_x - sxc) * 0.5
    offy = (node_size_y - syc) * 0.5
    area = node_size_x * node_size_y
    ratio = area / (sxc * syc)
    mean_area = jnp.mean(area[:_NMOV]) * 10.0
    row_h = jnp.min(node_size_y[:_NMOV]) * 2.0
    macro = (area[:_NMOV] > mean_area) & (node_size_y[:_NMOV] > row_h)
    ratio_mov = jnp.where(macro, _TD, ratio[:_NMOV])

    mov = slice(0, _NMOV)
    fix = slice(_NMOV, _NMOV + _NTERM)
    fil = slice(_NMOV + _NTERM, _NTOT)

    def _padm(a):
        return jnp.pad(a, (0, _NMF_PAD - _NMF))

    xm = _padm(jnp.concatenate([pos_x[mov] + offx[mov], pos_x[fil] + offx[fil]]))
    ym = _padm(jnp.concatenate([pos_y[mov] + offy[mov], pos_y[fil] + offy[fil]]))
    sxm = _padm(jnp.concatenate([sxc[mov], sxc[fil]]))
    sym = _padm(jnp.concatenate([syc[mov], syc[fil]]))
    wm = _padm(jnp.concatenate([ratio_mov, ratio[fil]]))

    def _padf(a):
        return jnp.pad(a, (0, _NF_PAD - _NTERM))

    xfp = _padf(pos_x[fix])
    yfp = _padf(pos_y[fix])
    sxfp = _padf(node_size_x[fix])
    syfp = _padf(node_size_y[fix])
    wfp = _padf(jnp.full((_NTERM,), _TD, jnp.float32))

    zeros = jnp.zeros((_NBINS,), jnp.float32)

    maps = _sc_density_maps(xm, ym, sxm, sym, wm,
                            xfp, yfp, sxfp, syfp, wfp, zeros)

    out_spec = pl.BlockSpec((1, 1), lambda: (0, 0))
    cost, maxd = pl.pallas_call(
        _reduce_kernel,
        in_specs=[pl.BlockSpec((_NC * _NB, _NB), lambda: (0, 0))],
        out_specs=[out_spec, out_spec],
        out_shape=[jax.ShapeDtypeStruct((1, 1), jnp.float32),
                   jax.ShapeDtypeStruct((1, 1), jnp.float32)],
    )(maps.reshape(_NC * _NB, _NB))
    return (cost.reshape(()), maxd.reshape(()))


# speedup vs baseline: 126.8951x; 1.1698x over previous
import functools
import math

import jax
import jax.numpy as jnp
from jax import lax
from jax.experimental import pallas as pl
from jax.experimental.pallas import tpu as pltpu
from jax.experimental.pallas import tpu_sc as plsc

_NMOV = 500000
_NTERM = 10000
_NFIL = 190000
_NTOT = _NMOV + _NTERM + _NFIL
_NMF = _NMOV + _NFIL
_NB = 512
_NBINS = _NB * _NB
_XL, _YL, _XH, _YH = 0.0, 0.0, 1000.0, 1000.0
_BS = (_XH - _XL) / _NB
_INV_BS = 1.0 / _BS
_TD = 0.9
_SQRT2 = math.sqrt(2.0)
_BIN_AREA = _BS * _BS

_NC = 2
_NS = 16
_NW = _NC * _NS
_L = 16

_CH = 1024
_CPW = 22
_MV_PER_W = _CPW * _CH
_NMF_PAD = _NW * _MV_PER_W

_FX_PER_W = 384
_NF_PAD = _NW * _FX_PER_W
_FXV = _FX_PER_W // _L

_SLICE = _NBINS // _NS


def _win_overlap(p, size, k_off):
    bi = jnp.clip((p * _INV_BS).astype(jnp.int32), 0, _NB - 1)
    bw = bi + k_off
    valid = bw < _NB
    bc = jnp.minimum(bw, _NB - 1)
    blo = bc.astype(jnp.float32) * _BS
    ov = jnp.minimum(p + size, blo + _BS) - jnp.maximum(p, blo)
    ov = jnp.where(valid, jnp.maximum(ov, 0.0), 0.0)
    return ov, bc


def _sc_body(xm, ym, sxm, sym, wm, xf, yf, sxf, syf, wf, zeros_hbm,
             out_maps,
             xb, yb, sxb, syb, wb, vals0, idx0, vals1, idx1,
             fxb, fyb, fsxb, fsyb, fwb, fvals, fidx2,
             sem, fsem, dmap):
    cid = lax.axis_index("c")
    sid = lax.axis_index("s")
    wid = cid * _NS + sid

    pltpu.sync_copy(zeros_hbm.at[pl.ds(sid * _SLICE, _SLICE)],
                    dmap.at[pl.ds(sid * _SLICE, _SLICE)])
    plsc.subcore_barrier()

    fbase = wid * _FX_PER_W
    pltpu.sync_copy(xf.at[pl.ds(fbase, _FX_PER_W)], fxb)
    pltpu.sync_copy(yf.at[pl.ds(fbase, _FX_PER_W)], fyb)
    pltpu.sync_copy(sxf.at[pl.ds(fbase, _FX_PER_W)], fsxb)
    pltpu.sync_copy(syf.at[pl.ds(fbase, _FX_PER_W)], fsyb)
    pltpu.sync_copy(wf.at[pl.ds(fbase, _FX_PER_W)], fwb)

    def fvec_body(v, _):
        o = v * _L
        x = fxb[pl.ds(o, _L)]
        y = fyb[pl.ds(o, _L)]
        sx = fsxb[pl.ds(o, _L)]
        sy = fsyb[pl.ds(o, _L)]
        w = fwb[pl.ds(o, _L)]
        pxs = []
        pys = []
        for k in range(8):
            px, bxc = _win_overlap(x, sx, k)
            pxs.append((px * w, bxc * _NB))
            py, byc = _win_overlap(y, sy, k)
            pys.append((py, byc))
        for i in range(8):
            pxw, bx5 = pxs[i]
            for j in range(8):
                py, byc = pys[j]
                p = (i * 8 + j) * _FX_PER_W + o
                fvals[pl.ds(p, _L)] = pxw * py
                fidx2[pl.ds(p, _L)] = bx5 + byc
        return _

    lax.fori_loop(0, _FXV, fvec_body, None)
    pltpu.async_copy(fvals, dmap.at[fidx2], fsem, add=True)

    def mv_chunk(ch, vals_b, idx_b):
        base = wid * _MV_PER_W + ch * _CH
        pltpu.sync_copy(xm.at[pl.ds(base, _CH)], xb)
        pltpu.sync_copy(ym.at[pl.ds(base, _CH)], yb)
        pltpu.sync_copy(sxm.at[pl.ds(base, _CH)], sxb)
        pltpu.sync_copy(sym.at[pl.ds(base, _CH)], syb)
        pltpu.sync_copy(wm.at[pl.ds(base, _CH)], wb)

        def vec_body(v, _):
            o = v * _L
            x = xb[pl.ds(o, _L)]
            y = yb[pl.ds(o, _L)]
            sx = sxb[pl.ds(o, _L)]
            sy = syb[pl.ds(o, _L)]
            w = wb[pl.ds(o, _L)]
            pxs = []
            pys = []
            for k in range(3):
                px, bxc = _win_overlap(x, sx, k)
                pxs.append((px * w, bxc * _NB))
                py, byc = _win_overlap(y, sy, k)
                pys.append((py, byc))
            for i in range(3):
                pxw, bx5 = pxs[i]
                for j in range(3):
                    py, byc = pys[j]
                    k9 = i * 3 + j
                    vals_b[pl.ds(k9 * _CH + o, _L)] = pxw * py
                    idx_b[pl.ds(k9 * _CH + o, _L)] = bx5 + byc
            return _

        lax.fori_loop(0, _CH // _L, vec_body, None)

    def pair_body(t, _):
        mv_chunk(2 * t, vals0, idx0)

        @pl.when(t > 0)
        def _drain_b():
            pltpu.make_async_copy(vals1, dmap.at[idx1], sem).wait()

        pltpu.async_copy(vals0, dmap.at[idx0], sem, add=True)
        mv_chunk(2 * t + 1, vals1, idx1)
        pltpu.make_async_copy(vals0, dmap.at[idx0], sem).wait()
        pltpu.async_copy(vals1, dmap.at[idx1], sem, add=True)
        return _

    lax.fori_loop(0, _CPW // 2, pair_body, None)
    pltpu.make_async_copy(vals1, dmap.at[idx1], sem).wait()
    pltpu.make_async_copy(fvals, dmap.at[fidx2], fsem).wait()

    plsc.subcore_barrier()
    pltpu.sync_copy(dmap.at[pl.ds(sid * _SLICE, _SLICE)],
                    out_maps.at[cid, pl.ds(sid * _SLICE, _SLICE)])


def _sc_density_maps(xm, ym, sxm, sym, wm, xf, yf, sxf, syf, wf, zeros):
    mesh = plsc.VectorSubcoreMesh(core_axis_name="c", subcore_axis_name="s",
                                  num_cores=_NC, num_subcores=_NS)
    f = pl.kernel(
        _sc_body,
        out_type=jax.ShapeDtypeStruct((_NC, _NBINS), jnp.float32),
        mesh=mesh,
        scratch_types=[
            pltpu.VMEM((_CH,), jnp.float32),
            pltpu.VMEM((_CH,), jnp.float32),
            pltpu.VMEM((_CH,), jnp.float32),
            pltpu.VMEM((_CH,), jnp.float32),
            pltpu.VMEM((_CH,), jnp.float32),
            pltpu.VMEM((9 * _CH,), jnp.float32),
            pltpu.VMEM((9 * _CH,), jnp.int32),
            pltpu.VMEM((9 * _CH,), jnp.float32),
            pltpu.VMEM((9 * _CH,), jnp.int32),
            pltpu.VMEM((_FX_PER_W,), jnp.float32),
            pltpu.VMEM((_FX_PER_W,), jnp.float32),
            pltpu.VMEM((_FX_PER_W,), jnp.float32),
            pltpu.VMEM((_FX_PER_W,), jnp.float32),
            pltpu.VMEM((_FX_PER_W,), jnp.float32),
            pltpu.VMEM((64 * _FX_PER_W,), jnp.float32),
            pltpu.VMEM((64 * _FX_PER_W,), jnp.int32),
            pltpu.SemaphoreType.DMA,
            pltpu.SemaphoreType.DMA,
            pltpu.VMEM_SHARED((_NBINS,), jnp.float32),
        ],
    )
    return f(xm, ym, sxm, sym, wm, xf, yf, sxf, syf, wf, zeros)


def _reduce_kernel(maps_ref, cost_ref, maxd_ref):
    a = maps_ref[:, :]
    d = a[:_NB, :] + a[_NB:, :]
    cost_ref[:, :] = jnp.sum(jnp.maximum(d - _TD * _BIN_AREA, 0.0),
                             keepdims=True)
    maxd_ref[:, :] = jnp.max(d, keepdims=True) / _BIN_AREA


def kernel(pos, node_size_x, node_size_y, bin_center_x, bin_center_y):
    pos_x = pos[:_NTOT]
    pos_y = pos[_NTOT:]
    sxc = jnp.maximum(node_size_x, _BS * _SQRT2)
    syc = jnp.maximum(node_size_y, _BS * _SQRT2)
    offx = (node_size_x - sxc) * 0.5
    offy = (node_size_y - syc) * 0.5
    area = node_size_x * node_size_y
    ratio = area / (sxc * syc)
    mean_area = jnp.mean(area[:_NMOV]) * 10.0
    row_h = jnp.min(node_size_y[:_NMOV]) * 2.0
    macro = (area[:_NMOV] > mean_area) & (node_size_y[:_NMOV] > row_h)
    ratio_mov = jnp.where(macro, _TD, ratio[:_NMOV])

    mov = slice(0, _NMOV)
    fix = slice(_NMOV, _NMOV + _NTERM)
    fil = slice(_NMOV + _NTERM, _NTOT)

    def _padm(a):
        return jnp.pad(a, (0, _NMF_PAD - _NMF))

    xm = _padm(jnp.concatenate([pos_x[mov] + offx[mov], pos_x[fil] + offx[fil]]))
    ym = _padm(jnp.concatenate([pos_y[mov] + offy[mov], pos_y[fil] + offy[fil]]))
    sxm = _padm(jnp.concatenate([sxc[mov], sxc[fil]]))
    sym = _padm(jnp.concatenate([syc[mov], syc[fil]]))
    wm = _padm(jnp.concatenate([ratio_mov, ratio[fil]]))

    def _padf(a):
        return jnp.pad(a, (0, _NF_PAD - _NTERM))

    xfp = _padf(pos_x[fix])
    yfp = _padf(pos_y[fix])
    sxfp = _padf(node_size_x[fix])
    syfp = _padf(node_size_y[fix])
    wfp = _padf(jnp.full((_NTERM,), _TD, jnp.float32))

    zeros = jnp.zeros((_NBINS,), jnp.float32)

    maps = _sc_density_maps(xm, ym, sxm, sym, wm,
                            xfp, yfp, sxfp, syfp, wfp, zeros)

    out_spec = pl.BlockSpec((1, 1), lambda: (0, 0))
    cost, maxd = pl.pallas_call(
        _reduce_kernel,
        in_specs=[pl.BlockSpec((_NC * _NB, _NB), lambda: (0, 0))],
        out_specs=[out_spec, out_spec],
        out_shape=[jax.ShapeDtypeStruct((1, 1), jnp.float32),
                   jax.ShapeDtypeStruct((1, 1), jnp.float32)],
    )(maps.reshape(_NC * _NB, _NB))
    return (cost.reshape(()), maxd.reshape(()))


# speedup vs baseline: 281.7762x; 2.2205x over previous
import functools
import math

import jax
import jax.numpy as jnp
from jax import lax
from jax.experimental import pallas as pl
from jax.experimental.pallas import tpu as pltpu
from jax.experimental.pallas import tpu_sc as plsc

_NMOV = 500000
_NTERM = 10000
_NFIL = 190000
_NTOT = _NMOV + _NTERM + _NFIL
_NMF = _NMOV + _NFIL
_NB = 512
_NBINS = _NB * _NB
_XL, _YL, _XH, _YH = 0.0, 0.0, 1000.0, 1000.0
_BS = (_XH - _XL) / _NB
_INV_BS = 1.0 / _BS
_TD = 0.9
_SQRT2 = math.sqrt(2.0)
_BIN_AREA = _BS * _BS

_NC = 2
_NS = 16
_NW = _NC * _NS
_L = 16

_CH = 1024
_CPW = 22
_MV_PER_W = _CPW * _CH
_NMF_PAD = _NW * _MV_PER_W

_FX_PER_W = 384
_NF_PAD = _NW * _FX_PER_W
_FXV = _FX_PER_W // _L

_SLICE = _NBINS // _NS
_IGNORED = -1


def _win_overlap(p, size, k_off):
    bi = jnp.clip((p * _INV_BS).astype(jnp.int32), 0, _NB - 1)
    bw = bi + k_off
    valid = bw < _NB
    bc = jnp.minimum(bw, _NB - 1)
    blo = bc.astype(jnp.float32) * _BS
    ov = jnp.minimum(p + size, blo + _BS) - jnp.maximum(p, blo)
    ov = jnp.where(valid, jnp.maximum(ov, 0.0), 0.0)
    return ov, bc


def _sc_body(xm, ym, sxm, sym, wm, xf, yf, sxf, syf, wf, zeros_hbm,
             out_maps,
             xb, yb, sxb, syb, wb, vals0, idx0, vals1, idx1,
             fxb, fyb, fsxb, fsyb, fwb, fvals, fidx2,
             sem, fsem, dmap):
    cid = lax.axis_index("c")
    sid = lax.axis_index("s")
    wid = cid * _NS + sid

    pltpu.sync_copy(zeros_hbm.at[pl.ds(sid * _SLICE, _SLICE)],
                    dmap.at[pl.ds(sid * _SLICE, _SLICE)])
    plsc.subcore_barrier()

    fbase = wid * _FX_PER_W
    pltpu.sync_copy(xf.at[pl.ds(fbase, _FX_PER_W)], fxb)
    pltpu.sync_copy(yf.at[pl.ds(fbase, _FX_PER_W)], fyb)
    pltpu.sync_copy(sxf.at[pl.ds(fbase, _FX_PER_W)], fsxb)
    pltpu.sync_copy(syf.at[pl.ds(fbase, _FX_PER_W)], fsyb)
    pltpu.sync_copy(wf.at[pl.ds(fbase, _FX_PER_W)], fwb)

    def fvec_body(v, _):
        o = v * _L
        x = fxb[pl.ds(o, _L)]
        y = fyb[pl.ds(o, _L)]
        sx = fsxb[pl.ds(o, _L)]
        sy = fsyb[pl.ds(o, _L)]
        w = fwb[pl.ds(o, _L)]
        pxs = []
        pys = []
        for k in range(8):
            px, bxc = _win_overlap(x, sx, k)
            pxs.append((px * w, bxc * _NB))
            py, byc = _win_overlap(y, sy, k)
            pys.append((py, byc))
        for i in range(8):
            pxw, bx5 = pxs[i]
            for j in range(8):
                py, byc = pys[j]
                p = (i * 8 + j) * _FX_PER_W + o
                val = pxw * py
                fvals[pl.ds(p, _L)] = val
                fidx2[pl.ds(p, _L)] = jnp.where(val != 0.0, bx5 + byc,
                                                _IGNORED)
        return _

    lax.fori_loop(0, _FXV, fvec_body, None)
    pltpu.async_copy(fvals, dmap.at[plsc.Indices(fidx2, ignored_value=_IGNORED)], fsem, add=True)

    def mv_chunk(ch, vals_b, idx_b):
        base = wid * _MV_PER_W + ch * _CH
        pltpu.sync_copy(xm.at[pl.ds(base, _CH)], xb)
        pltpu.sync_copy(ym.at[pl.ds(base, _CH)], yb)
        pltpu.sync_copy(sxm.at[pl.ds(base, _CH)], sxb)
        pltpu.sync_copy(sym.at[pl.ds(base, _CH)], syb)
        pltpu.sync_copy(wm.at[pl.ds(base, _CH)], wb)

        def vec_body(v, _):
            o = v * _L
            x = xb[pl.ds(o, _L)]
            y = yb[pl.ds(o, _L)]
            sx = sxb[pl.ds(o, _L)]
            sy = syb[pl.ds(o, _L)]
            w = wb[pl.ds(o, _L)]
            pxs = []
            pys = []
            for k in range(3):
                px, bxc = _win_overlap(x, sx, k)
                pxs.append((px * w, bxc * _NB))
                py, byc = _win_overlap(y, sy, k)
                pys.append((py, byc))
            for i in range(3):
                pxw, bx5 = pxs[i]
                for j in range(3):
                    py, byc = pys[j]
                    k9 = i * 3 + j
                    val = pxw * py
                    vals_b[pl.ds(k9 * _CH + o, _L)] = val
                    idx_b[pl.ds(k9 * _CH + o, _L)] = jnp.where(
                        val != 0.0, bx5 + byc, _IGNORED)
            return _

        lax.fori_loop(0, _CH // _L, vec_body, None)

    def pair_body(t, _):
        mv_chunk(2 * t, vals0, idx0)

        @pl.when(t > 0)
        def _drain_b():
            pltpu.make_async_copy(vals1, dmap.at[plsc.Indices(idx1, ignored_value=_IGNORED)], sem).wait()

        pltpu.async_copy(vals0, dmap.at[plsc.Indices(idx0, ignored_value=_IGNORED)], sem, add=True)
        mv_chunk(2 * t + 1, vals1, idx1)
        pltpu.make_async_copy(vals0, dmap.at[plsc.Indices(idx0, ignored_value=_IGNORED)], sem).wait()
        pltpu.async_copy(vals1, dmap.at[plsc.Indices(idx1, ignored_value=_IGNORED)], sem, add=True)
        return _

    lax.fori_loop(0, _CPW // 2, pair_body, None)
    pltpu.make_async_copy(vals1, dmap.at[plsc.Indices(idx1, ignored_value=_IGNORED)], sem).wait()
    pltpu.make_async_copy(fvals, dmap.at[plsc.Indices(fidx2, ignored_value=_IGNORED)], fsem).wait()

    plsc.subcore_barrier()
    pltpu.sync_copy(dmap.at[pl.ds(sid * _SLICE, _SLICE)],
                    out_maps.at[cid, pl.ds(sid * _SLICE, _SLICE)])


def _sc_density_maps(xm, ym, sxm, sym, wm, xf, yf, sxf, syf, wf, zeros):
    mesh = plsc.VectorSubcoreMesh(core_axis_name="c", subcore_axis_name="s",
                                  num_cores=_NC, num_subcores=_NS)
    f = pl.kernel(
        _sc_body,
        out_type=jax.ShapeDtypeStruct((_NC, _NBINS), jnp.float32),
        mesh=mesh,
        scratch_types=[
            pltpu.VMEM((_CH,), jnp.float32),
            pltpu.VMEM((_CH,), jnp.float32),
            pltpu.VMEM((_CH,), jnp.float32),
            pltpu.VMEM((_CH,), jnp.float32),
            pltpu.VMEM((_CH,), jnp.float32),
            pltpu.VMEM((9 * _CH,), jnp.float32),
            pltpu.VMEM((9 * _CH,), jnp.int32),
            pltpu.VMEM((9 * _CH,), jnp.float32),
            pltpu.VMEM((9 * _CH,), jnp.int32),
            pltpu.VMEM((_FX_PER_W,), jnp.float32),
            pltpu.VMEM((_FX_PER_W,), jnp.float32),
            pltpu.VMEM((_FX_PER_W,), jnp.float32),
            pltpu.VMEM((_FX_PER_W,), jnp.float32),
            pltpu.VMEM((_FX_PER_W,), jnp.float32),
            pltpu.VMEM((64 * _FX_PER_W,), jnp.float32),
            pltpu.VMEM((64 * _FX_PER_W,), jnp.int32),
            pltpu.SemaphoreType.DMA,
            pltpu.SemaphoreType.DMA,
            pltpu.VMEM_SHARED((_NBINS,), jnp.float32),
        ],
    )
    return f(xm, ym, sxm, sym, wm, xf, yf, sxf, syf, wf, zeros)


def _reduce_kernel(maps_ref, cost_ref, maxd_ref):
    a = maps_ref[:, :]
    d = a[:_NB, :] + a[_NB:, :]
    cost_ref[:, :] = jnp.sum(jnp.maximum(d - _TD * _BIN_AREA, 0.0),
                             keepdims=True)
    maxd_ref[:, :] = jnp.max(d, keepdims=True) / _BIN_AREA


def kernel(pos, node_size_x, node_size_y, bin_center_x, bin_center_y):
    pos_x = pos[:_NTOT]
    pos_y = pos[_NTOT:]
    sxc = jnp.maximum(node_size_x, _BS * _SQRT2)
    syc = jnp.maximum(node_size_y, _BS * _SQRT2)
    offx = (node_size_x - sxc) * 0.5
    offy = (node_size_y - syc) * 0.5
    area = node_size_x * node_size_y
    ratio = area / (sxc * syc)
    mean_area = jnp.mean(area[:_NMOV]) * 10.0
    row_h = jnp.min(node_size_y[:_NMOV]) * 2.0
    macro = (area[:_NMOV] > mean_area) & (node_size_y[:_NMOV] > row_h)
    ratio_mov = jnp.where(macro, _TD, ratio[:_NMOV])

    mov = slice(0, _NMOV)
    fix = slice(_NMOV, _NMOV + _NTERM)
    fil = slice(_NMOV + _NTERM, _NTOT)

    def _padm(a):
        return jnp.pad(a, (0, _NMF_PAD - _NMF))

    xm = _padm(jnp.concatenate([pos_x[mov] + offx[mov], pos_x[fil] + offx[fil]]))
    ym = _padm(jnp.concatenate([pos_y[mov] + offy[mov], pos_y[fil] + offy[fil]]))
    sxm = _padm(jnp.concatenate([sxc[mov], sxc[fil]]))
    sym = _padm(jnp.concatenate([syc[mov], syc[fil]]))
    wm = _padm(jnp.concatenate([ratio_mov, ratio[fil]]))

    def _padf(a):
        return jnp.pad(a, (0, _NF_PAD - _NTERM))

    xfp = _padf(pos_x[fix])
    yfp = _padf(pos_y[fix])
    sxfp = _padf(node_size_x[fix])
    syfp = _padf(node_size_y[fix])
    wfp = _padf(jnp.full((_NTERM,), _TD, jnp.float32))

    zeros = jnp.zeros((_NBINS,), jnp.float32)

    maps = _sc_density_maps(xm, ym, sxm, sym, wm,
                            xfp, yfp, sxfp, syfp, wfp, zeros)

    out_spec = pl.BlockSpec((1, 1), lambda: (0, 0))
    cost, maxd = pl.pallas_call(
        _reduce_kernel,
        in_specs=[pl.BlockSpec((_NC * _NB, _NB), lambda: (0, 0))],
        out_specs=[out_spec, out_spec],
        out_shape=[jax.ShapeDtypeStruct((1, 1), jnp.float32),
                   jax.ShapeDtypeStruct((1, 1), jnp.float32)],
    )(maps.reshape(_NC * _NB, _NB))
    return (cost.reshape(()), maxd.reshape(()))


# speedup vs baseline: 287.1523x; 1.0191x over previous
import functools
import math

import jax
import jax.numpy as jnp
from jax import lax
from jax.experimental import pallas as pl
from jax.experimental.pallas import tpu as pltpu
from jax.experimental.pallas import tpu_sc as plsc

_NMOV = 500000
_NTERM = 10000
_NFIL = 190000
_NTOT = _NMOV + _NTERM + _NFIL
_NMF = _NMOV + _NFIL
_NB = 512
_NBINS = _NB * _NB
_XL, _YL, _XH, _YH = 0.0, 0.0, 1000.0, 1000.0
_BS = (_XH - _XL) / _NB
_INV_BS = 1.0 / _BS
_TD = 0.9
_SQRT2 = math.sqrt(2.0)
_BIN_AREA = _BS * _BS

_NC = 2
_NS = 16
_NW = _NC * _NS
_L = 16

_CH = 1024
_CPW = 22
_MV_PER_W = _CPW * _CH
_NMF_PAD = _NW * _MV_PER_W

_FX_PER_W = 384
_NF_PAD = _NW * _FX_PER_W
_FXV = _FX_PER_W // _L

_SLICE = _NBINS // _NS
_IGNORED = -1


def _axis_windows(p, size, n_k):
    pe = p + size
    bi = jnp.clip((p * _INV_BS).astype(jnp.int32), 0, _NB - 1)
    blo0 = bi.astype(jnp.float32) * _BS
    out = []
    for k in range(n_k):
        blo = blo0 + (k * _BS)
        ov = jnp.maximum(
            jnp.minimum(pe, blo + _BS) - jnp.maximum(p, blo), 0.0)
        out.append((ov, jnp.minimum(bi + k, _NB - 1)))
    return out


def _sc_body(xm, ym, sxm, sym, wm, xf, yf, sxf, syf, wf, zeros_hbm,
             out_maps,
             xb, yb, sxb, syb, wb, vals0, idx0, vals1, idx1,
             fxb, fyb, fsxb, fsyb, fwb, fvals, fidx2,
             sem, fsem, dmap):
    cid = lax.axis_index("c")
    sid = lax.axis_index("s")
    wid = cid * _NS + sid

    pltpu.sync_copy(zeros_hbm.at[pl.ds(sid * _SLICE, _SLICE)],
                    dmap.at[pl.ds(sid * _SLICE, _SLICE)])
    plsc.subcore_barrier()

    fbase = wid * _FX_PER_W
    pltpu.sync_copy(xf.at[pl.ds(fbase, _FX_PER_W)], fxb)
    pltpu.sync_copy(yf.at[pl.ds(fbase, _FX_PER_W)], fyb)
    pltpu.sync_copy(sxf.at[pl.ds(fbase, _FX_PER_W)], fsxb)
    pltpu.sync_copy(syf.at[pl.ds(fbase, _FX_PER_W)], fsyb)
    pltpu.sync_copy(wf.at[pl.ds(fbase, _FX_PER_W)], fwb)

    def fvec_body(v, _):
        o = v * _L
        x = fxb[pl.ds(o, _L)]
        y = fyb[pl.ds(o, _L)]
        sx = fsxb[pl.ds(o, _L)]
        sy = fsyb[pl.ds(o, _L)]
        w = fwb[pl.ds(o, _L)]
        pxs = [(px * w, bxc * _NB) for px, bxc in _axis_windows(x, sx, 8)]
        pys = _axis_windows(y, sy, 8)
        for i in range(8):
            pxw, bx5 = pxs[i]
            for j in range(8):
                py, byc = pys[j]
                p = (i * 8 + j) * _FX_PER_W + o
                val = pxw * py
                fvals[pl.ds(p, _L)] = val
                fidx2[pl.ds(p, _L)] = jnp.where(val != 0.0, bx5 + byc,
                                                _IGNORED)
        return _

    lax.fori_loop(0, _FXV, fvec_body, None)
    pltpu.async_copy(fvals, dmap.at[plsc.Indices(fidx2, ignored_value=_IGNORED)], fsem, add=True)

    def mv_chunk(ch, vals_b, idx_b):
        base = wid * _MV_PER_W + ch * _CH
        pltpu.sync_copy(xm.at[pl.ds(base, _CH)], xb)
        pltpu.sync_copy(ym.at[pl.ds(base, _CH)], yb)
        pltpu.sync_copy(sxm.at[pl.ds(base, _CH)], sxb)
        pltpu.sync_copy(sym.at[pl.ds(base, _CH)], syb)
        pltpu.sync_copy(wm.at[pl.ds(base, _CH)], wb)

        def vec_body(v, _):
            o = v * _L
            x = xb[pl.ds(o, _L)]
            y = yb[pl.ds(o, _L)]
            sx = sxb[pl.ds(o, _L)]
            sy = syb[pl.ds(o, _L)]
            w = wb[pl.ds(o, _L)]
            pxs = [(px * w, bxc * _NB)
                   for px, bxc in _axis_windows(x, sx, 3)]
            pys = _axis_windows(y, sy, 3)
            for i in range(3):
                pxw, bx5 = pxs[i]
                for j in range(3):
                    py, byc = pys[j]
                    k9 = i * 3 + j
                    val = pxw * py
                    vals_b[pl.ds(k9 * _CH + o, _L)] = val
                    idx_b[pl.ds(k9 * _CH + o, _L)] = jnp.where(
                        val != 0.0, bx5 + byc, _IGNORED)
            return _

        lax.fori_loop(0, _CH // _L, vec_body, None)

    def pair_body(t, _):
        mv_chunk(2 * t, vals0, idx0)

        @pl.when(t > 0)
        def _drain_b():
            pltpu.make_async_copy(vals1, dmap.at[plsc.Indices(idx1, ignored_value=_IGNORED)], sem).wait()

        pltpu.async_copy(vals0, dmap.at[plsc.Indices(idx0, ignored_value=_IGNORED)], sem, add=True)
        mv_chunk(2 * t + 1, vals1, idx1)
        pltpu.make_async_copy(vals0, dmap.at[plsc.Indices(idx0, ignored_value=_IGNORED)], sem).wait()
        pltpu.async_copy(vals1, dmap.at[plsc.Indices(idx1, ignored_value=_IGNORED)], sem, add=True)
        return _

    lax.fori_loop(0, _CPW // 2, pair_body, None)
    pltpu.make_async_copy(vals1, dmap.at[plsc.Indices(idx1, ignored_value=_IGNORED)], sem).wait()
    pltpu.make_async_copy(fvals, dmap.at[plsc.Indices(fidx2, ignored_value=_IGNORED)], fsem).wait()

    plsc.subcore_barrier()
    pltpu.sync_copy(dmap.at[pl.ds(sid * _SLICE, _SLICE)],
                    out_maps.at[cid, pl.ds(sid * _SLICE, _SLICE)])


def _sc_density_maps(xm, ym, sxm, sym, wm, xf, yf, sxf, syf, wf, zeros):
    mesh = plsc.VectorSubcoreMesh(core_axis_name="c", subcore_axis_name="s",
                                  num_cores=_NC, num_subcores=_NS)
    f = pl.kernel(
        _sc_body,
        out_type=jax.ShapeDtypeStruct((_NC, _NBINS), jnp.float32),
        mesh=mesh,
        scratch_types=[
            pltpu.VMEM((_CH,), jnp.float32),
            pltpu.VMEM((_CH,), jnp.float32),
            pltpu.VMEM((_CH,), jnp.float32),
            pltpu.VMEM((_CH,), jnp.float32),
            pltpu.VMEM((_CH,), jnp.float32),
            pltpu.VMEM((9 * _CH,), jnp.float32),
            pltpu.VMEM((9 * _CH,), jnp.int32),
            pltpu.VMEM((9 * _CH,), jnp.float32),
            pltpu.VMEM((9 * _CH,), jnp.int32),
            pltpu.VMEM((_FX_PER_W,), jnp.float32),
            pltpu.VMEM((_FX_PER_W,), jnp.float32),
            pltpu.VMEM((_FX_PER_W,), jnp.float32),
            pltpu.VMEM((_FX_PER_W,), jnp.float32),
            pltpu.VMEM((_FX_PER_W,), jnp.float32),
            pltpu.VMEM((64 * _FX_PER_W,), jnp.float32),
            pltpu.VMEM((64 * _FX_PER_W,), jnp.int32),
            pltpu.SemaphoreType.DMA,
            pltpu.SemaphoreType.DMA,
            pltpu.VMEM_SHARED((_NBINS,), jnp.float32),
        ],
    )
    return f(xm, ym, sxm, sym, wm, xf, yf, sxf, syf, wf, zeros)


def _reduce_kernel(maps_ref, cost_ref, maxd_ref):
    a = maps_ref[:, :]
    d = a[:_NB, :] + a[_NB:, :]
    cost_ref[:, :] = jnp.sum(jnp.maximum(d - _TD * _BIN_AREA, 0.0),
                             keepdims=True)
    maxd_ref[:, :] = jnp.max(d, keepdims=True) / _BIN_AREA


def kernel(pos, node_size_x, node_size_y, bin_center_x, bin_center_y):
    pos_x = pos[:_NTOT]
    pos_y = pos[_NTOT:]
    sxc = jnp.maximum(node_size_x, _BS * _SQRT2)
    syc = jnp.maximum(node_size_y, _BS * _SQRT2)
    offx = (node_size_x - sxc) * 0.5
    offy = (node_size_y - syc) * 0.5
    area = node_size_x * node_size_y
    ratio = area / (sxc * syc)
    mean_area = jnp.mean(area[:_NMOV]) * 10.0
    row_h = jnp.min(node_size_y[:_NMOV]) * 2.0
    macro = (area[:_NMOV] > mean_area) & (node_size_y[:_NMOV] > row_h)
    ratio_mov = jnp.where(macro, _TD, ratio[:_NMOV])

    mov = slice(0, _NMOV)
    fix = slice(_NMOV, _NMOV + _NTERM)
    fil = slice(_NMOV + _NTERM, _NTOT)

    def _padm(a):
        return jnp.pad(a, (0, _NMF_PAD - _NMF))

    xm = _padm(jnp.concatenate([pos_x[mov] + offx[mov], pos_x[fil] + offx[fil]]))
    ym = _padm(jnp.concatenate([pos_y[mov] + offy[mov], pos_y[fil] + offy[fil]]))
    sxm = _padm(jnp.concatenate([sxc[mov], sxc[fil]]))
    sym = _padm(jnp.concatenate([syc[mov], syc[fil]]))
    wm = _padm(jnp.concatenate([ratio_mov, ratio[fil]]))

    def _padf(a):
        return jnp.pad(a, (0, _NF_PAD - _NTERM))

    xfp = _padf(pos_x[fix])
    yfp = _padf(pos_y[fix])
    sxfp = _padf(node_size_x[fix])
    syfp = _padf(node_size_y[fix])
    wfp = _padf(jnp.full((_NTERM,), _TD, jnp.float32))

    zeros = jnp.zeros((_NBINS,), jnp.float32)

    maps = _sc_density_maps(xm, ym, sxm, sym, wm,
                            xfp, yfp, sxfp, syfp, wfp, zeros)

    out_spec = pl.BlockSpec((1, 1), lambda: (0, 0))
    cost, maxd = pl.pallas_call(
        _reduce_kernel,
        in_specs=[pl.BlockSpec((_NC * _NB, _NB), lambda: (0, 0))],
        out_specs=[out_spec, out_spec],
        out_shape=[jax.ShapeDtypeStruct((1, 1), jnp.float32),
                   jax.ShapeDtypeStruct((1, 1), jnp.float32)],
    )(maps.reshape(_NC * _NB, _NB))
    return (cost.reshape(()), maxd.reshape(()))


# speedup vs baseline: 347.2410x; 1.2093x over previous
import math

import jax
import jax.numpy as jnp
from jax import lax
from jax.experimental import pallas as pl
from jax.experimental.pallas import tpu as pltpu
from jax.experimental.pallas import tpu_sc as plsc

_NMOV = 500000
_NTERM = 10000
_NFIL = 190000
_NTOT = _NMOV + _NTERM + _NFIL
_NB = 512
_NBINS = _NB * _NB
_XL, _YL, _XH, _YH = 0.0, 0.0, 1000.0, 1000.0
_BS = (_XH - _XL) / _NB
_INV_BS = 1.0 / _BS
_TD = 0.9
_SQRT2 = math.sqrt(2.0)
_KS = _BS * _SQRT2
_BIN_AREA = _BS * _BS

_NC = 2
_NS = 16
_NW = _NC * _NS
_L = 16

_CH = 1024
_MV_CPW = 16
_MV_PER_W = _MV_CPW * _CH
_FIL_BASE = _NMOV + _NTERM
_FIL_CPW = 6
_FIL_PER_W = _FIL_CPW * _CH
_FX_BASE = _NMOV
_FX_PER_W = 320
_FXV = _FX_PER_W // _L

_SZ_PAD = _FIL_BASE + _NW * _FIL_PER_W
_POS_PAD = _NTOT + _SZ_PAD

_SLICE = _NBINS // _NS
_IGNORED = -1


def _axis_windows(p, size, n_k):
    pe = p + size
    bi = jnp.clip((p * _INV_BS).astype(jnp.int32), 0, _NB - 1)
    blo0 = bi.astype(jnp.float32) * _BS
    out = []
    for k in range(n_k):
        blo = blo0 + (k * _BS)
        ov = jnp.maximum(
            jnp.minimum(pe, blo + _BS) - jnp.maximum(p, blo), 0.0)
        out.append((ov, jnp.minimum(bi + k, _NB - 1)))
    return out


def _sc_body(pos, nsx, nsy, m10v, rh2v, zeros_hbm,
             out_maps,
             xb, yb, sxb, syb, vals0, idx0, vals1, idx1,
             fvals, fidx, m10b, rh2b,
             sem, fsem, dmap):
    cid = lax.axis_index("c")
    sid = lax.axis_index("s")
    wid = cid * _NS + sid
    lane = lax.iota(jnp.int32, _L)

    pltpu.sync_copy(zeros_hbm.at[pl.ds(sid * _SLICE, _SLICE)],
                    dmap.at[pl.ds(sid * _SLICE, _SLICE)])
    pltpu.sync_copy(m10v, m10b)
    pltpu.sync_copy(rh2v, rh2b)
    plsc.subcore_barrier()

    fbase = _FX_BASE + wid * _FX_PER_W
    pltpu.sync_copy(pos.at[pl.ds(fbase, _FX_PER_W)], xb.at[pl.ds(0, _FX_PER_W)])
    pltpu.sync_copy(pos.at[pl.ds(_NTOT + fbase, _FX_PER_W)],
                    yb.at[pl.ds(0, _FX_PER_W)])
    pltpu.sync_copy(nsx.at[pl.ds(fbase, _FX_PER_W)],
                    sxb.at[pl.ds(0, _FX_PER_W)])
    pltpu.sync_copy(nsy.at[pl.ds(fbase, _FX_PER_W)],
                    syb.at[pl.ds(0, _FX_PER_W)])

    def fvec_body(v, _):
        o = v * _L
        x = xb[pl.ds(o, _L)]
        y = yb[pl.ds(o, _L)]
        sx = sxb[pl.ds(o, _L)]
        sy = syb[pl.ds(o, _L)]
        w = jnp.where(lane < _FX_BASE + _NTERM - fbase - o, _TD, 0.0)
        pxs = [(px * w, bxc * _NB) for px, bxc in _axis_windows(x, sx, 8)]
        pys = _axis_windows(y, sy, 8)
        for i in range(8):
            pxw, bx5 = pxs[i]
            for j in range(8):
                py, byc = pys[j]
                p = (i * 8 + j) * _FX_PER_W + o
                val = pxw * py
                fvals[pl.ds(p, _L)] = val
                fidx[pl.ds(p, _L)] = jnp.where(val != 0.0, bx5 + byc,
                                               _IGNORED)
        return _

    lax.fori_loop(0, _FXV, fvec_body, None)
    pltpu.async_copy(fvals, dmap.at[plsc.Indices(fidx, ignored_value=_IGNORED)],
                     fsem, add=True)

    def mv_chunk(base, limit, with_macro, vals_b, idx_b):
        pltpu.sync_copy(pos.at[pl.ds(base, _CH)], xb)
        pltpu.sync_copy(pos.at[pl.ds(_NTOT + base, _CH)], yb)
        pltpu.sync_copy(nsx.at[pl.ds(base, _CH)], sxb)
        pltpu.sync_copy(nsy.at[pl.ds(base, _CH)], syb)
        m10 = m10b[pl.ds(0, _L)]
        rh2 = rh2b[pl.ds(0, _L)]

        def vec_body(v, _):
            o = v * _L
            x = xb[pl.ds(o, _L)]
            y = yb[pl.ds(o, _L)]
            nx = sxb[pl.ds(o, _L)]
            ny = syb[pl.ds(o, _L)]
            sxc = jnp.maximum(nx, _KS)
            syc = jnp.maximum(ny, _KS)
            xs = x + (nx - sxc) * 0.5
            ys = y + (ny - syc) * 0.5
            area = nx * ny
            w = area / (sxc * syc)
            if with_macro:
                w = jnp.where((area > m10) & (ny > rh2), _TD, w)
            w = jnp.where(lane < limit - base - o, w, 0.0)
            pxs = [(px * w, bxc * _NB)
                   for px, bxc in _axis_windows(xs, sxc, 3)]
            pys = _axis_windows(ys, syc, 3)
            for i in range(3):
                pxw, bx5 = pxs[i]
                for j in range(3):
                    py, byc = pys[j]
                    k9 = i * 3 + j
                    val = pxw * py
                    vals_b[pl.ds(k9 * _CH + o, _L)] = val
                    idx_b[pl.ds(k9 * _CH + o, _L)] = jnp.where(
                        val != 0.0, bx5 + byc, _IGNORED)
            return _

        lax.fori_loop(0, _CH // _L, vec_body, None)

    def run_phase(pair0, npairs, wbase, limit, with_macro):
        def pair_body(t, _):
            base = wbase + (2 * t) * _CH
            mv_chunk(base, limit, with_macro, vals0, idx0)

            @pl.when(t > pair0)
            def _drain_b():
                pltpu.make_async_copy(
                    vals1, dmap.at[plsc.Indices(idx1, ignored_value=_IGNORED)],
                    sem).wait()

            pltpu.async_copy(
                vals0, dmap.at[plsc.Indices(idx0, ignored_value=_IGNORED)],
                sem, add=True)
            mv_chunk(base + _CH, limit, with_macro, vals1, idx1)
            pltpu.make_async_copy(
                vals0, dmap.at[plsc.Indices(idx0, ignored_value=_IGNORED)],
                sem).wait()
            pltpu.async_copy(
                vals1, dmap.at[plsc.Indices(idx1, ignored_value=_IGNORED)],
                sem, add=True)
            return _

        lax.fori_loop(pair0, pair0 + npairs, pair_body, None)
        pltpu.make_async_copy(
            vals1, dmap.at[plsc.Indices(idx1, ignored_value=_IGNORED)],
            sem).wait()

    run_phase(0, _MV_CPW // 2, wid * _MV_PER_W, _NMOV, True)
    run_phase(0, _FIL_CPW // 2, _FIL_BASE + wid * _FIL_PER_W, _NTOT, False)
    pltpu.make_async_copy(
        fvals, dmap.at[plsc.Indices(fidx, ignored_value=_IGNORED)],
        fsem).wait()

    plsc.subcore_barrier()
    pltpu.sync_copy(dmap.at[pl.ds(sid * _SLICE, _SLICE)],
                    out_maps.at[cid, pl.ds(sid * _SLICE, _SLICE)])


def _sc_density_maps(pos, nsx, nsy, m10v, rh2v, zeros):
    mesh = plsc.VectorSubcoreMesh(core_axis_name="c", subcore_axis_name="s",
                                  num_cores=_NC, num_subcores=_NS)
    f = pl.kernel(
        _sc_body,
        out_type=jax.ShapeDtypeStruct((_NC, _NBINS), jnp.float32),
        mesh=mesh,
        scratch_types=[
            pltpu.VMEM((_CH,), jnp.float32),
            pltpu.VMEM((_CH,), jnp.float32),
            pltpu.VMEM((_CH,), jnp.float32),
            pltpu.VMEM((_CH,), jnp.float32),
            pltpu.VMEM((9 * _CH,), jnp.float32),
            pltpu.VMEM((9 * _CH,), jnp.int32),
            pltpu.VMEM((9 * _CH,), jnp.float32),
            pltpu.VMEM((9 * _CH,), jnp.int32),
            pltpu.VMEM((64 * _FX_PER_W,), jnp.float32),
            pltpu.VMEM((64 * _FX_PER_W,), jnp.int32),
            pltpu.VMEM((_L,), jnp.float32),
            pltpu.VMEM((_L,), jnp.float32),
            pltpu.SemaphoreType.DMA,
            pltpu.SemaphoreType.DMA,
            pltpu.VMEM_SHARED((_NBINS,), jnp.float32),
        ],
    )
    return f(pos, nsx, nsy, m10v, rh2v, zeros)


def _reduce_kernel(maps_ref, cost_ref, maxd_ref):
    a = maps_ref[:, :]
    d = a[:_NB, :] + a[_NB:, :]
    cost_ref[:, :] = jnp.sum(jnp.maximum(d - _TD * _BIN_AREA, 0.0),
                             keepdims=True)
    maxd_ref[:, :] = jnp.max(d, keepdims=True) / _BIN_AREA


def kernel(pos, node_size_x, node_size_y, bin_center_x, bin_center_y):
    area_mov = node_size_x[:_NMOV] * node_size_y[:_NMOV]
    m10 = jnp.mean(area_mov) * 10.0
    rh2 = jnp.min(node_size_y[:_NMOV]) * 2.0
    m10v = jnp.full((_L,), m10, jnp.float32)
    rh2v = jnp.full((_L,), rh2, jnp.float32)

    pos_pad = jnp.pad(pos, (0, _POS_PAD - 2 * _NTOT))
    nsx_pad = jnp.pad(node_size_x, (0, _SZ_PAD - _NTOT))
    nsy_pad = jnp.pad(node_size_y, (0, _SZ_PAD - _NTOT))
    zeros = jnp.zeros((_NBINS,), jnp.float32)

    maps = _sc_density_maps(pos_pad, nsx_pad, nsy_pad, m10v, rh2v, zeros)

    out_spec = pl.BlockSpec((1, 1), lambda: (0, 0))
    cost, maxd = pl.pallas_call(
        _reduce_kernel,
        in_specs=[pl.BlockSpec((_NC * _NB, _NB), lambda: (0, 0))],
        out_specs=[out_spec, out_spec],
        out_shape=[jax.ShapeDtypeStruct((1, 1), jnp.float32),
                   jax.ShapeDtypeStruct((1, 1), jnp.float32)],
    )(maps.reshape(_NC * _NB, _NB))
    return (cost.reshape(()), maxd.reshape(()))
